# Initial kernel scaffold; baseline (speedup 1.0000x reference)
import jax
import jax.numpy as jnp
from jax.experimental import pallas as pl


def kernel(node_features, edge_index, edge_features, graph_index, lin0_W, lin0_b, en1_W, en1_b, en2_W, en2_b, conv_b, gru_Wih, gru_Whh, gru_bih, gru_bhh, lstm_Wih, lstm_Whh, lstm_bih, lstm_bhh, fc1_W, fc1_b, fc2_W, fc2_b):
    raise NotImplementedError("write your pallas kernel here")


# speedup vs baseline: 1.4936x; 1.4936x over previous
import functools

import jax
import jax.numpy as jnp
from jax import lax
from jax.experimental import pallas as pl
from jax.experimental.pallas import tpu as pltpu
from jax.experimental.pallas import tpu_sc as plsc

N = 10000
E = 160000
F_IN = 128
DIM = 16
NG = 64
EH = 128

NC, NS, L = 2, 16, 16
NW = NC * NS
EPT = 5120
EP = NW * EPT
CH = 128
NCHUNK = EPT // CH
GRP = 8
NP = 10240
RPT = NP // NS
EB = 2048
NEB = EP // EB
NB = 1024
NNB = NP // NB

_mesh = plsc.VectorSubcoreMesh(
    core_axis_name="c", subcore_axis_name="s", num_cores=NC, num_subcores=NS)


@functools.partial(
    pl.kernel,
    out_type=jax.ShapeDtypeStruct((EP, DIM), jnp.float32),
    mesh=_mesh,
    compiler_params=pltpu.CompilerParams(use_tc_tiling_on_sc=False),
    scratch_types=[
        pltpu.VMEM((NCHUNK, CH), jnp.int32),
        pltpu.VMEM((EPT, DIM), jnp.float32),
        pltpu.SemaphoreType.DMA,
    ],
)
def _sc_gather(table_hbm, idx_hbm, out_hbm, idx_v, rows_v, sem):
    c = lax.axis_index("c")
    s = lax.axis_index("s")
    wid = s * NC + c
    base = wid * EPT
    pltpu.sync_copy(idx_hbm.at[wid], idx_v)

    def grp_body(g, carry):
        hs = []
        for b in range(GRP):
            j = g * GRP + b
            hs.append(
                pltpu.async_copy(
                    table_hbm.at[idx_v.at[j]],
                    rows_v.at[pl.ds(j * CH, CH)],
                    sem,
                )
            )
        for h in hs:
            h.wait()
        return carry

    lax.fori_loop(0, NCHUNK // GRP, grp_body, 0)
    pltpu.sync_copy(rows_v, out_hbm.at[pl.ds(base, EPT)])


@functools.partial(
    pl.kernel,
    out_type=jax.ShapeDtypeStruct((NC, NP, DIM), jnp.float32),
    mesh=_mesh,
    compiler_params=pltpu.CompilerParams(use_tc_tiling_on_sc=False),
    scratch_types=[
        pltpu.VMEM((NCHUNK, CH), jnp.int32),
        pltpu.VMEM((EPT, DIM), jnp.float32),
        pltpu.VMEM_SHARED((NP, DIM), jnp.float32),
        pltpu.SemaphoreType.DMA,
    ],
)
def _sc_scatter_add(msg_hbm, idx_hbm, zeros_hbm, out_hbm, idx_v, msg_v, acc_sh, sem):
    c = lax.axis_index("c")
    s = lax.axis_index("s")
    wid = s * NC + c
    base = wid * EPT

    @pl.when(s == 0)
    def _():
        pltpu.sync_copy(zeros_hbm, acc_sh)

    plsc.subcore_barrier()
    pltpu.sync_copy(idx_hbm.at[wid], idx_v)
    pltpu.sync_copy(msg_hbm.at[pl.ds(base, EPT)], msg_v)

    def grp_body(g, carry):
        hs = []
        for b in range(GRP):
            j = g * GRP + b
            hs.append(
                pltpu.async_copy(
                    msg_v.at[pl.ds(j * CH, CH)],
                    acc_sh.at[idx_v.at[j]],
                    sem,
                    add=True,
                )
            )
        for h in hs:
            h.wait()
        return carry

    lax.fori_loop(0, NCHUNK // GRP, grp_body, 0)
    plsc.subcore_barrier()
    pltpu.sync_copy(acc_sh.at[pl.ds(s * RPT, RPT)], msg_v.at[pl.ds(0, RPT)])
    pltpu.sync_copy(msg_v.at[pl.ds(0, RPT)], out_hbm.at[c, pl.ds(s * RPT, RPT)])


def _mm(a, b):
    return jnp.matmul(a, b, precision=lax.Precision.HIGHEST)


def _edge_net_body(ef_ref, w1_ref, b1_ref, w2_ref, b2_ref, ew_ref):
    h = jnp.maximum(_mm(ef_ref[...], w1_ref[...]) + b1_ref[...], 0.0)
    ew_ref[...] = _mm(h, w2_ref[...]) + b2_ref[...]


def _lin0_body(nf_ref, w_ref, b_ref, out_ref):
    out_ref[...] = jnp.maximum(_mm(nf_ref[...], w_ref[...]) + b_ref[...], 0.0)


def _msg_body(xg_ref, ew_ref, k1_ref, k2_ref, msg_ref):
    xrep = _mm(xg_ref[...], k1_ref[...])
    p = xrep * ew_ref[...]
    m = _mm(p, k2_ref[...])
    eglob = pl.program_id(0) * EB + lax.broadcasted_iota(jnp.int32, (EB, 1), 0)
    msg_ref[...] = jnp.where(eglob < E, m, 0.0)


def _gru_body(agg0_ref, agg1_ref, deg0_ref, deg1_ref, h_ref, convb_ref,
              wir_ref, wiz_ref, win_ref, whr_ref, whz_ref, whn_ref,
              br_ref, bz_ref, bin_ref, bhn_ref, out_ref):
    deg = jnp.maximum(deg0_ref[...] + deg1_ref[...], 1.0)
    agg = (agg0_ref[...] + agg1_ref[...]) / deg + convb_ref[...]
    m = jnp.maximum(agg, 0.0)
    h = h_ref[...]
    r = jax.nn.sigmoid(_mm(m, wir_ref[...]) + _mm(h, whr_ref[...]) + br_ref[...])
    z = jax.nn.sigmoid(_mm(m, wiz_ref[...]) + _mm(h, whz_ref[...]) + bz_ref[...])
    n = jnp.tanh(_mm(m, win_ref[...]) + bin_ref[...]
                 + r * (_mm(h, whn_ref[...]) + bhn_ref[...]))
    out_ref[...] = (1.0 - z) * n + z * h


def _set2set_body(m_ref, gcol_ref, grow_ref,
                  wq_i_ref, wr_i_ref, wh_i_ref, b_i_ref,
                  wq_f_ref, wr_f_ref, wh_f_ref, b_f_ref,
                  wq_g_ref, wr_g_ref, wh_g_ref, b_g_ref,
                  wq_o_ref, wr_o_ref, wh_o_ref, b_o_ref,
                  fc1q_ref, fc1r_ref, fc1b_ref, fc2_ref, fc2b_ref,
                  out_ref):
    M = m_ref[...]
    gcol = gcol_ref[...]
    grow = grow_ref[...]
    onehot = (gcol == lax.broadcasted_iota(jnp.int32, (NP, NG), 1)
              ).astype(jnp.float32)
    onehot_t = (grow == lax.broadcasted_iota(jnp.int32, (NG, NP), 0)
                ).astype(jnp.float32)

    qq = jnp.zeros((NG, DIM), jnp.float32)
    rv = jnp.zeros((NG, DIM), jnp.float32)
    hh = jnp.zeros((NG, DIM), jnp.float32)
    cc = jnp.zeros((NG, DIM), jnp.float32)
    for _ in range(3):
        ig = jax.nn.sigmoid(_mm(qq, wq_i_ref[...]) + _mm(rv, wr_i_ref[...])
                            + _mm(hh, wh_i_ref[...]) + b_i_ref[...])
        fg = jax.nn.sigmoid(_mm(qq, wq_f_ref[...]) + _mm(rv, wr_f_ref[...])
                            + _mm(hh, wh_f_ref[...]) + b_f_ref[...])
        gg = jnp.tanh(_mm(qq, wq_g_ref[...]) + _mm(rv, wr_g_ref[...])
                      + _mm(hh, wh_g_ref[...]) + b_g_ref[...])
        og = jax.nn.sigmoid(_mm(qq, wq_o_ref[...]) + _mm(rv, wr_o_ref[...])
                            + _mm(hh, wh_o_ref[...]) + b_o_ref[...])
        cc = fg * cc + ig * gg
        hh = og * jnp.tanh(cc)
        q = hh
        qn = _mm(onehot, q)
        e = jnp.sum(M * qn, axis=1, keepdims=True)
        masked = jnp.where(onehot > 0.0, e, -jnp.inf)
        emax = jnp.max(masked, axis=0, keepdims=True)
        emax = jnp.where(emax == -jnp.inf, 0.0, emax)
        emax_n = jnp.sum(onehot * emax, axis=1, keepdims=True)
        ee = jnp.exp(e - emax_n)
        denom = jnp.sum(onehot * ee, axis=0, keepdims=True)
        denom_n = jnp.sum(onehot * denom, axis=1, keepdims=True)
        a = ee / (denom_n + 1e-16)
        rv = _mm(onehot_t, a * M)
        qq = q
    o1 = jnp.maximum(_mm(qq, fc1q_ref[...]) + _mm(rv, fc1r_ref[...]) + fc1b_ref[...], 0.0)
    out_ref[...] = _mm(o1, fc2_ref[...]) + fc2b_ref[...]


def kernel(node_features, edge_index, edge_features, graph_index,
           lin0_W, lin0_b, en1_W, en1_b, en2_W, en2_b, conv_b,
           gru_Wih, gru_Whh, gru_bih, gru_bhh,
           lstm_Wih, lstm_Whh, lstm_bih, lstm_bhh,
           fc1_W, fc1_b, fc2_W, fc2_b):
    f32 = jnp.float32
    src_r = jnp.pad(edge_index[0], (0, EP - E)).reshape(NW, NCHUNK, CH)
    dst_r = jnp.pad(edge_index[1], (0, EP - E)).reshape(NW, NCHUNK, CH)
    ef_pad = jnp.pad(edge_features, ((0, EP - E), (0, 16 - 11)))
    nf_pad = jnp.pad(node_features, ((0, NP - N), (0, 0)))
    gidx_pad = jnp.pad(graph_index, (0, NP - N), constant_values=NG)
    gcol = gidx_pad.reshape(NP, 1)
    grow = gidx_pad.reshape(1, NP)
    ones_msg = jnp.pad(jnp.ones((E, DIM), f32), ((0, EP - E), (0, 0)))
    zeros_np = jnp.zeros((NP, DIM), f32)
    k1 = jnp.kron(jnp.eye(DIM, dtype=f32), jnp.ones((1, DIM), f32))
    k2 = jnp.kron(jnp.ones((DIM, 1), f32), jnp.eye(DIM, dtype=f32))

    en1_Wt = jnp.pad(en1_W.T, ((0, 16 - 11), (0, 0)))
    en2_Wt = en2_W.T
    b1 = en1_b.reshape(1, EH)
    b2 = en2_b.reshape(1, DIM * DIM)
    lin0_Wt = lin0_W.T
    lin0_br = lin0_b.reshape(1, DIM)
    convb = conv_b.reshape(1, DIM)

    wir = gru_Wih[0:DIM].T
    wiz = gru_Wih[DIM:2 * DIM].T
    win = gru_Wih[2 * DIM:3 * DIM].T
    whr = gru_Whh[0:DIM].T
    whz = gru_Whh[DIM:2 * DIM].T
    whn = gru_Whh[2 * DIM:3 * DIM].T
    br = (gru_bih[0:DIM] + gru_bhh[0:DIM]).reshape(1, DIM)
    bz = (gru_bih[DIM:2 * DIM] + gru_bhh[DIM:2 * DIM]).reshape(1, DIM)
    bin_ = gru_bih[2 * DIM:3 * DIM].reshape(1, DIM)
    bhn = gru_bhh[2 * DIM:3 * DIM].reshape(1, DIM)

    def lstm_slices(g):
        lo = g * DIM
        wq = lstm_Wih[lo:lo + DIM, 0:DIM].T
        wr = lstm_Wih[lo:lo + DIM, DIM:2 * DIM].T
        wh = lstm_Whh[lo:lo + DIM].T
        b = (lstm_bih[lo:lo + DIM] + lstm_bhh[lo:lo + DIM]).reshape(1, DIM)
        return wq, wr, wh, b

    lstm_w = [w for g in range(4) for w in lstm_slices(g)]
    fc1q = fc1_W[:, 0:DIM].T
    fc1r = fc1_W[:, DIM:2 * DIM].T
    fc1b = fc1_b.reshape(1, DIM)
    fc2t = fc2_W.T
    fc2b = fc2_b.reshape(1, 1)

    ew = pl.pallas_call(
        _edge_net_body,
        grid=(NEB,),
        in_specs=[
            pl.BlockSpec((EB, 16), lambda i: (i, 0)),
            pl.BlockSpec((16, EH), lambda i: (0, 0)),
            pl.BlockSpec((1, EH), lambda i: (0, 0)),
            pl.BlockSpec((EH, DIM * DIM), lambda i: (0, 0)),
            pl.BlockSpec((1, DIM * DIM), lambda i: (0, 0)),
        ],
        out_specs=pl.BlockSpec((EB, DIM * DIM), lambda i: (i, 0)),
        out_shape=jax.ShapeDtypeStruct((EP, DIM * DIM), f32),
    )(ef_pad, en1_Wt, b1, en2_Wt, b2)

    h = pl.pallas_call(
        _lin0_body,
        grid=(NNB,),
        in_specs=[
            pl.BlockSpec((NB, F_IN), lambda i: (i, 0)),
            pl.BlockSpec((F_IN, DIM), lambda i: (0, 0)),
            pl.BlockSpec((1, DIM), lambda i: (0, 0)),
        ],
        out_specs=pl.BlockSpec((NB, DIM), lambda i: (i, 0)),
        out_shape=jax.ShapeDtypeStruct((NP, DIM), f32),
    )(nf_pad, lin0_Wt, lin0_br)

    degp = _sc_scatter_add(ones_msg, dst_r, zeros_np)

    msg_call = pl.pallas_call(
        _msg_body,
        grid=(NEB,),
        in_specs=[
            pl.BlockSpec((EB, DIM), lambda i: (i, 0)),
            pl.BlockSpec((EB, DIM * DIM), lambda i: (i, 0)),
            pl.BlockSpec((DIM, DIM * DIM), lambda i: (0, 0)),
            pl.BlockSpec((DIM * DIM, DIM), lambda i: (0, 0)),
        ],
        out_specs=pl.BlockSpec((EB, DIM), lambda i: (i, 0)),
        out_shape=jax.ShapeDtypeStruct((EP, DIM), f32),
    )

    _nblk = pl.BlockSpec((NB, DIM), lambda i: (i, 0))
    _wblk = pl.BlockSpec((DIM, DIM), lambda i: (0, 0))
    _bblk = pl.BlockSpec((1, DIM), lambda i: (0, 0))
    gru_call = pl.pallas_call(
        _gru_body,
        grid=(NNB,),
        in_specs=[_nblk] * 5 + [_bblk] + [_wblk] * 6 + [_bblk] * 4,
        out_specs=_nblk,
        out_shape=jax.ShapeDtypeStruct((NP, DIM), f32),
    )

    for _ in range(3):
        xg = _sc_gather(h, src_r)
        msg = msg_call(xg, ew, k1, k2)
        aggp = _sc_scatter_add(msg, dst_r, zeros_np)
        h = gru_call(aggp[0], aggp[1], degp[0], degp[1], h, convb,
                     wir, wiz, win, whr, whz, whn, br, bz, bin_, bhn)

    pred = pl.pallas_call(
        _set2set_body,
        out_shape=jax.ShapeDtypeStruct((NG, 1), f32),
    )(h, gcol, grow, *lstm_w, fc1q, fc1r, fc1b, fc2t, fc2b)
    return pred.reshape(-1)


# speedup vs baseline: 2.2814x; 1.5275x over previous
import functools

import jax
import jax.numpy as jnp
from jax import lax
from jax.experimental import pallas as pl
from jax.experimental.pallas import tpu as pltpu
from jax.experimental.pallas import tpu_sc as plsc

N = 10000
E = 160000
F_IN = 128
DIM = 16
NG = 64
EH = 128

NC, NS, L = 2, 16, 16
NW = NC * NS
EPT = 5120
EP = NW * EPT
CH = 128
NCHUNK = EPT // CH
GRP = 8
NP = 10240
RPT = NP // NS
EB = 2048
NEB = EP // EB
NB = 1024
NNB = NP // NB

_mesh = plsc.VectorSubcoreMesh(
    core_axis_name="c", subcore_axis_name="s", num_cores=NC, num_subcores=NS)


@functools.partial(
    pl.kernel,
    out_type=jax.ShapeDtypeStruct((EP, DIM), jnp.float32),
    mesh=_mesh,
    compiler_params=pltpu.CompilerParams(use_tc_tiling_on_sc=False),
    scratch_types=[
        pltpu.VMEM((NCHUNK, CH), jnp.int32),
        pltpu.VMEM((EPT, DIM), jnp.float32),
        pltpu.SemaphoreType.DMA,
    ],
)
def _sc_gather(table_hbm, idx_hbm, out_hbm, idx_v, rows_v, sem):
    c = lax.axis_index("c")
    s = lax.axis_index("s")
    wid = s * NC + c
    base = wid * EPT
    pltpu.sync_copy(idx_hbm.at[wid], idx_v)

    def grp_body(g, carry):
        hs = []
        for b in range(GRP):
            j = g * GRP + b
            hs.append(
                pltpu.async_copy(
                    table_hbm.at[idx_v.at[j]],
                    rows_v.at[pl.ds(j * CH, CH)],
                    sem,
                )
            )
        for h in hs:
            h.wait()
        return carry

    lax.fori_loop(0, NCHUNK // GRP, grp_body, 0)
    pltpu.sync_copy(rows_v, out_hbm.at[pl.ds(base, EPT)])


@functools.partial(
    pl.kernel,
    out_type=jax.ShapeDtypeStruct((NC, NP, DIM), jnp.float32),
    mesh=_mesh,
    compiler_params=pltpu.CompilerParams(use_tc_tiling_on_sc=False),
    scratch_types=[
        pltpu.VMEM((NCHUNK, CH), jnp.int32),
        pltpu.VMEM((EPT, DIM), jnp.float32),
        pltpu.VMEM_SHARED((NP, DIM), jnp.float32),
        pltpu.SemaphoreType.DMA,
    ],
)
def _sc_scatter_add(msg_hbm, idx_hbm, zeros_hbm, out_hbm, idx_v, msg_v, acc_sh, sem):
    c = lax.axis_index("c")
    s = lax.axis_index("s")
    wid = s * NC + c
    base = wid * EPT

    @pl.when(s == 0)
    def _():
        pltpu.sync_copy(zeros_hbm, acc_sh)

    plsc.subcore_barrier()
    pltpu.sync_copy(idx_hbm.at[wid], idx_v)
    pltpu.sync_copy(msg_hbm.at[pl.ds(base, EPT)], msg_v)

    def grp_body(g, carry):
        hs = []
        for b in range(GRP):
            j = g * GRP + b
            hs.append(
                pltpu.async_copy(
                    msg_v.at[pl.ds(j * CH, CH)],
                    acc_sh.at[idx_v.at[j]],
                    sem,
                    add=True,
                )
            )
        for h in hs:
            h.wait()
        return carry

    lax.fori_loop(0, NCHUNK // GRP, grp_body, 0)
    plsc.subcore_barrier()
    pltpu.sync_copy(acc_sh.at[pl.ds(s * RPT, RPT)], msg_v.at[pl.ds(0, RPT)])
    pltpu.sync_copy(msg_v.at[pl.ds(0, RPT)], out_hbm.at[c, pl.ds(s * RPT, RPT)])


def _mm(a, b):
    return jnp.matmul(a, b, precision=lax.Precision.HIGHEST)


def _bsplit(a):
    hi = a.astype(jnp.bfloat16).astype(jnp.float32)
    return hi, a - hi


def _mm3(a, b):
    ah, al = _bsplit(a)
    bh, bl = _bsplit(b)
    return jnp.dot(ah, bh) + (jnp.dot(al, bh) + jnp.dot(ah, bl))


def _mm2(a, b):
    ah, al = _bsplit(a)
    return jnp.dot(ah, b) + jnp.dot(al, b)


def _edge_net_body(ef_ref, w1_ref, b1_ref, w2_ref, b2_ref, ew_ref):
    h = jnp.maximum(_mm(ef_ref[...], w1_ref[...]) + b1_ref[...], 0.0)
    ew_ref[...] = _mm(h, w2_ref[...]) + b2_ref[...]


def _lin0_body(nf_ref, w_ref, b_ref, out_ref):
    out_ref[...] = jnp.maximum(_mm(nf_ref[...], w_ref[...]) + b_ref[...], 0.0)


def _msg_body(xg_ref, ew_ref, k2_ref, msg_ref):
    xg = xg_ref[...]
    xrep = jnp.concatenate([xg] * 16, axis=1)
    p = xrep * ew_ref[...]
    m = _mm2(p, k2_ref[...])
    eglob = pl.program_id(0) * EB + lax.broadcasted_iota(jnp.int32, (EB, 1), 0)
    msg_ref[...] = jnp.where(eglob < E, m, 0.0)


def _gru_body(agg0_ref, agg1_ref, deg0_ref, deg1_ref, h_ref, convb_ref,
              wir_ref, wiz_ref, win_ref, whr_ref, whz_ref, whn_ref,
              br_ref, bz_ref, bin_ref, bhn_ref, out_ref):
    deg = jnp.maximum(deg0_ref[...] + deg1_ref[...], 1.0)
    agg = (agg0_ref[...] + agg1_ref[...]) / deg + convb_ref[...]
    m = jnp.maximum(agg, 0.0)
    h = h_ref[...]
    r = jax.nn.sigmoid(_mm(m, wir_ref[...]) + _mm(h, whr_ref[...]) + br_ref[...])
    z = jax.nn.sigmoid(_mm(m, wiz_ref[...]) + _mm(h, whz_ref[...]) + bz_ref[...])
    n = jnp.tanh(_mm(m, win_ref[...]) + bin_ref[...]
                 + r * (_mm(h, whn_ref[...]) + bhn_ref[...]))
    out_ref[...] = (1.0 - z) * n + z * h


def _set2set_body(m_ref, gcol_ref, grow_ref,
                  wq_i_ref, wr_i_ref, wh_i_ref, b_i_ref,
                  wq_f_ref, wr_f_ref, wh_f_ref, b_f_ref,
                  wq_g_ref, wr_g_ref, wh_g_ref, b_g_ref,
                  wq_o_ref, wr_o_ref, wh_o_ref, b_o_ref,
                  fc1q_ref, fc1r_ref, fc1b_ref, fc2_ref, fc2b_ref,
                  out_ref):
    M = m_ref[...]
    gcol = gcol_ref[...]
    grow = grow_ref[...]
    onehot = (gcol == lax.broadcasted_iota(jnp.int32, (NP, NG), 1)
              ).astype(jnp.float32)
    onehot_t = (grow == lax.broadcasted_iota(jnp.int32, (NG, NP), 0)
                ).astype(jnp.float32)

    qq = jnp.zeros((NG, DIM), jnp.float32)
    rv = jnp.zeros((NG, DIM), jnp.float32)
    hh = jnp.zeros((NG, DIM), jnp.float32)
    cc = jnp.zeros((NG, DIM), jnp.float32)
    for _ in range(3):
        ig = jax.nn.sigmoid(_mm(qq, wq_i_ref[...]) + _mm(rv, wr_i_ref[...])
                            + _mm(hh, wh_i_ref[...]) + b_i_ref[...])
        fg = jax.nn.sigmoid(_mm(qq, wq_f_ref[...]) + _mm(rv, wr_f_ref[...])
                            + _mm(hh, wh_f_ref[...]) + b_f_ref[...])
        gg = jnp.tanh(_mm(qq, wq_g_ref[...]) + _mm(rv, wr_g_ref[...])
                      + _mm(hh, wh_g_ref[...]) + b_g_ref[...])
        og = jax.nn.sigmoid(_mm(qq, wq_o_ref[...]) + _mm(rv, wr_o_ref[...])
                            + _mm(hh, wh_o_ref[...]) + b_o_ref[...])
        cc = fg * cc + ig * gg
        hh = og * jnp.tanh(cc)
        q = hh
        qn = _mm(onehot, q)
        e = jnp.sum(M * qn, axis=1, keepdims=True)
        masked = jnp.where(onehot > 0.0, e, -jnp.inf)
        emax = jnp.max(masked, axis=0, keepdims=True)
        emax = jnp.where(emax == -jnp.inf, 0.0, emax)
        emax_n = jnp.sum(onehot * emax, axis=1, keepdims=True)
        ee = jnp.exp(e - emax_n)
        denom = jnp.sum(onehot * ee, axis=0, keepdims=True)
        denom_n = jnp.sum(onehot * denom, axis=1, keepdims=True)
        a = ee / (denom_n + 1e-16)
        rv = _mm(onehot_t, a * M)
        qq = q
    o1 = jnp.maximum(_mm(qq, fc1q_ref[...]) + _mm(rv, fc1r_ref[...]) + fc1b_ref[...], 0.0)
    out_ref[...] = _mm(o1, fc2_ref[...]) + fc2b_ref[...]


def kernel(node_features, edge_index, edge_features, graph_index,
           lin0_W, lin0_b, en1_W, en1_b, en2_W, en2_b, conv_b,
           gru_Wih, gru_Whh, gru_bih, gru_bhh,
           lstm_Wih, lstm_Whh, lstm_bih, lstm_bhh,
           fc1_W, fc1_b, fc2_W, fc2_b):
    f32 = jnp.float32
    src_r = jnp.pad(edge_index[0], (0, EP - E)).reshape(NW, NCHUNK, CH)
    dst_r = jnp.pad(edge_index[1], (0, EP - E)).reshape(NW, NCHUNK, CH)
    ef_pad = jnp.pad(edge_features, ((0, EP - E), (0, 16 - 11)))
    nf_pad = jnp.pad(node_features, ((0, NP - N), (0, 0)))
    gidx_pad = jnp.pad(graph_index, (0, NP - N), constant_values=NG)
    gcol = gidx_pad.reshape(NP, 1)
    grow = gidx_pad.reshape(1, NP)
    ones_msg = jnp.pad(jnp.ones((E, DIM), f32), ((0, EP - E), (0, 0)))
    zeros_np = jnp.zeros((NP, DIM), f32)
    perm = (jnp.arange(DIM * DIM) % DIM) * DIM + jnp.arange(DIM * DIM) // DIM
    k2p = jnp.kron(jnp.eye(DIM, dtype=f32), jnp.ones((DIM, 1), f32))

    en1_Wt = jnp.pad(en1_W.T, ((0, 16 - 11), (0, 0)))
    en2_Wt = en2_W[perm].T
    b1 = en1_b.reshape(1, EH)
    b2 = en2_b[perm].reshape(1, DIM * DIM)
    lin0_Wt = lin0_W.T
    lin0_br = lin0_b.reshape(1, DIM)
    convb = conv_b.reshape(1, DIM)

    wir = gru_Wih[0:DIM].T
    wiz = gru_Wih[DIM:2 * DIM].T
    win = gru_Wih[2 * DIM:3 * DIM].T
    whr = gru_Whh[0:DIM].T
    whz = gru_Whh[DIM:2 * DIM].T
    whn = gru_Whh[2 * DIM:3 * DIM].T
    br = (gru_bih[0:DIM] + gru_bhh[0:DIM]).reshape(1, DIM)
    bz = (gru_bih[DIM:2 * DIM] + gru_bhh[DIM:2 * DIM]).reshape(1, DIM)
    bin_ = gru_bih[2 * DIM:3 * DIM].reshape(1, DIM)
    bhn = gru_bhh[2 * DIM:3 * DIM].reshape(1, DIM)

    def lstm_slices(g):
        lo = g * DIM
        wq = lstm_Wih[lo:lo + DIM, 0:DIM].T
        wr = lstm_Wih[lo:lo + DIM, DIM:2 * DIM].T
        wh = lstm_Whh[lo:lo + DIM].T
        b = (lstm_bih[lo:lo + DIM] + lstm_bhh[lo:lo + DIM]).reshape(1, DIM)
        return wq, wr, wh, b

    lstm_w = [w for g in range(4) for w in lstm_slices(g)]
    fc1q = fc1_W[:, 0:DIM].T
    fc1r = fc1_W[:, DIM:2 * DIM].T
    fc1b = fc1_b.reshape(1, DIM)
    fc2t = fc2_W.T
    fc2b = fc2_b.reshape(1, 1)

    ew = pl.pallas_call(
        _edge_net_body,
        grid=(NEB,),
        in_specs=[
            pl.BlockSpec((EB, 16), lambda i: (i, 0)),
            pl.BlockSpec((16, EH), lambda i: (0, 0)),
            pl.BlockSpec((1, EH), lambda i: (0, 0)),
            pl.BlockSpec((EH, DIM * DIM), lambda i: (0, 0)),
            pl.BlockSpec((1, DIM * DIM), lambda i: (0, 0)),
        ],
        out_specs=pl.BlockSpec((EB, DIM * DIM), lambda i: (i, 0)),
        out_shape=jax.ShapeDtypeStruct((EP, DIM * DIM), f32),
    )(ef_pad, en1_Wt, b1, en2_Wt, b2)

    h = pl.pallas_call(
        _lin0_body,
        grid=(NNB,),
        in_specs=[
            pl.BlockSpec((NB, F_IN), lambda i: (i, 0)),
            pl.BlockSpec((F_IN, DIM), lambda i: (0, 0)),
            pl.BlockSpec((1, DIM), lambda i: (0, 0)),
        ],
        out_specs=pl.BlockSpec((NB, DIM), lambda i: (i, 0)),
        out_shape=jax.ShapeDtypeStruct((NP, DIM), f32),
    )(nf_pad, lin0_Wt, lin0_br)

    degp = _sc_scatter_add(ones_msg, dst_r, zeros_np)

    msg_call = pl.pallas_call(
        _msg_body,
        grid=(NEB,),
        in_specs=[
            pl.BlockSpec((EB, DIM), lambda i: (i, 0)),
            pl.BlockSpec((EB, DIM * DIM), lambda i: (i, 0)),
            pl.BlockSpec((DIM * DIM, DIM), lambda i: (0, 0)),
        ],
        out_specs=pl.BlockSpec((EB, DIM), lambda i: (i, 0)),
        out_shape=jax.ShapeDtypeStruct((EP, DIM), f32),
    )

    _nblk = pl.BlockSpec((NB, DIM), lambda i: (i, 0))
    _wblk = pl.BlockSpec((DIM, DIM), lambda i: (0, 0))
    _bblk = pl.BlockSpec((1, DIM), lambda i: (0, 0))
    gru_call = pl.pallas_call(
        _gru_body,
        grid=(NNB,),
        in_specs=[_nblk] * 5 + [_bblk] + [_wblk] * 6 + [_bblk] * 4,
        out_specs=_nblk,
        out_shape=jax.ShapeDtypeStruct((NP, DIM), f32),
    )

    for _ in range(3):
        xg = _sc_gather(h, src_r)
        msg = msg_call(xg, ew, k2p)
        aggp = _sc_scatter_add(msg, dst_r, zeros_np)
        h = gru_call(aggp[0], aggp[1], degp[0], degp[1], h, convb,
                     wir, wiz, win, whr, whz, whn, br, bz, bin_, bhn)

    pred = pl.pallas_call(
        _set2set_body,
        out_shape=jax.ShapeDtypeStruct((NG, 1), f32),
    )(h, gcol, grow, *lstm_w, fc1q, fc1r, fc1b, fc2t, fc2b)
    return pred.reshape(-1)


# speedup vs baseline: 2.2916x; 1.0045x over previous
import functools

import jax
import jax.numpy as jnp
from jax import lax
from jax.experimental import pallas as pl
from jax.experimental.pallas import tpu as pltpu
from jax.experimental.pallas import tpu_sc as plsc

N = 10000
E = 160000
F_IN = 128
DIM = 16
NG = 64
EH = 128

NC, NS, L = 2, 16, 16
NW = NC * NS
EPT = 5120
EP = NW * EPT
CH = 128
NCHUNK = EPT // CH
GRP = 8
NP = 10240
RPT = NP // NS
EB = 2048
NEB = EP // EB
NB = 1024
NNB = NP // NB

_mesh = plsc.VectorSubcoreMesh(
    core_axis_name="c", subcore_axis_name="s", num_cores=NC, num_subcores=NS)


@functools.partial(
    pl.kernel,
    out_type=jax.ShapeDtypeStruct((EP, DIM), jnp.float32),
    mesh=_mesh,
    compiler_params=pltpu.CompilerParams(use_tc_tiling_on_sc=False),
    scratch_types=[
        pltpu.VMEM((NCHUNK, CH), jnp.int32),
        pltpu.VMEM((EPT, DIM), jnp.float32),
        pltpu.SemaphoreType.DMA,
    ],
)
def _sc_gather(table_hbm, idx_hbm, out_hbm, idx_v, rows_v, sem):
    c = lax.axis_index("c")
    s = lax.axis_index("s")
    wid = s * NC + c
    base = wid * EPT
    pltpu.sync_copy(idx_hbm.at[wid], idx_v)

    def grp_body(g, carry):
        hs = []
        for b in range(GRP):
            j = g * GRP + b
            hs.append(
                pltpu.async_copy(
                    table_hbm.at[idx_v.at[j]],
                    rows_v.at[pl.ds(j * CH, CH)],
                    sem,
                )
            )
        for h in hs:
            h.wait()
        return carry

    lax.fori_loop(0, NCHUNK // GRP, grp_body, 0)
    pltpu.sync_copy(rows_v, out_hbm.at[pl.ds(base, EPT)])


@functools.partial(
    pl.kernel,
    out_type=jax.ShapeDtypeStruct((NC, NP, DIM), jnp.float32),
    mesh=_mesh,
    compiler_params=pltpu.CompilerParams(use_tc_tiling_on_sc=False),
    scratch_types=[
        pltpu.VMEM((NCHUNK, CH), jnp.int32),
        pltpu.VMEM((EPT, DIM), jnp.float32),
        pltpu.VMEM_SHARED((NP, DIM), jnp.float32),
        pltpu.SemaphoreType.DMA,
    ],
)
def _sc_scatter_add(msg_hbm, idx_hbm, zeros_hbm, out_hbm, idx_v, msg_v, acc_sh, sem):
    c = lax.axis_index("c")
    s = lax.axis_index("s")
    wid = s * NC + c
    base = wid * EPT

    @pl.when(s == 0)
    def _():
        pltpu.sync_copy(zeros_hbm, acc_sh)

    plsc.subcore_barrier()
    pltpu.sync_copy(idx_hbm.at[wid], idx_v)
    pltpu.sync_copy(msg_hbm.at[pl.ds(base, EPT)], msg_v)

    def grp_body(g, carry):
        hs = []
        for b in range(GRP):
            j = g * GRP + b
            hs.append(
                pltpu.async_copy(
                    msg_v.at[pl.ds(j * CH, CH)],
                    acc_sh.at[idx_v.at[j]],
                    sem,
                    add=True,
                )
            )
        for h in hs:
            h.wait()
        return carry

    lax.fori_loop(0, NCHUNK // GRP, grp_body, 0)
    plsc.subcore_barrier()
    pltpu.sync_copy(acc_sh.at[pl.ds(s * RPT, RPT)], msg_v.at[pl.ds(0, RPT)])
    pltpu.sync_copy(msg_v.at[pl.ds(0, RPT)], out_hbm.at[c, pl.ds(s * RPT, RPT)])


def _mm(a, b):
    return jnp.matmul(a, b, precision=lax.Precision.HIGHEST)


def _bsplit(a):
    hi = a.astype(jnp.bfloat16).astype(jnp.float32)
    return hi, a - hi


def _mm3(a, b):
    ah, al = _bsplit(a)
    bh, bl = _bsplit(b)
    return jnp.dot(ah, bh) + (jnp.dot(al, bh) + jnp.dot(ah, bl))


def _mm2(a, b):
    ah, al = _bsplit(a)
    return jnp.dot(ah, b) + jnp.dot(al, b)


def _edge_net_body(ef_ref, w1_ref, b1_ref, w2_ref, b2_ref, ew_ref):
    h = jnp.maximum(_mm(ef_ref[...], w1_ref[...]) + b1_ref[...], 0.0)
    ew_ref[...] = _mm(h, w2_ref[...]) + b2_ref[...]


def _lin0_body(nf_ref, w_ref, b_ref, out_ref):
    out_ref[...] = jnp.maximum(_mm(nf_ref[...], w_ref[...]) + b_ref[...], 0.0)


def _msg_body(xg_ref, ew_ref, k2_ref, msg_ref):
    xg = xg_ref[...]
    xrep = jnp.concatenate([xg] * 16, axis=1)
    p = xrep * ew_ref[...]
    msg_ref[...] = jnp.dot(p, k2_ref[...])


def _gru_body(agg0_ref, agg1_ref, deg0_ref, deg1_ref, h_ref, convb_ref,
              wir_ref, wiz_ref, win_ref, whr_ref, whz_ref, whn_ref,
              br_ref, bz_ref, bin_ref, bhn_ref, out_ref):
    deg = jnp.maximum(deg0_ref[...] + deg1_ref[...], 1.0)
    agg = (agg0_ref[...] + agg1_ref[...]) / deg + convb_ref[...]
    m = jnp.maximum(agg, 0.0)
    h = h_ref[...]
    r = jax.nn.sigmoid(_mm(m, wir_ref[...]) + _mm(h, whr_ref[...]) + br_ref[...])
    z = jax.nn.sigmoid(_mm(m, wiz_ref[...]) + _mm(h, whz_ref[...]) + bz_ref[...])
    n = jnp.tanh(_mm(m, win_ref[...]) + bin_ref[...]
                 + r * (_mm(h, whn_ref[...]) + bhn_ref[...]))
    out_ref[...] = (1.0 - z) * n + z * h


def _set2set_body(m_ref, gcol_ref, grow_ref,
                  wq_i_ref, wr_i_ref, wh_i_ref, b_i_ref,
                  wq_f_ref, wr_f_ref, wh_f_ref, b_f_ref,
                  wq_g_ref, wr_g_ref, wh_g_ref, b_g_ref,
                  wq_o_ref, wr_o_ref, wh_o_ref, b_o_ref,
                  fc1q_ref, fc1r_ref, fc1b_ref, fc2_ref, fc2b_ref,
                  out_ref):
    M = m_ref[...]
    gcol = gcol_ref[...]
    grow = grow_ref[...]
    onehot = (gcol == lax.broadcasted_iota(jnp.int32, (NP, NG), 1)
              ).astype(jnp.float32)
    onehot_t = (grow == lax.broadcasted_iota(jnp.int32, (NG, NP), 0)
                ).astype(jnp.float32)

    qq = jnp.zeros((NG, DIM), jnp.float32)
    rv = jnp.zeros((NG, DIM), jnp.float32)
    hh = jnp.zeros((NG, DIM), jnp.float32)
    cc = jnp.zeros((NG, DIM), jnp.float32)
    for _ in range(3):
        ig = jax.nn.sigmoid(_mm(qq, wq_i_ref[...]) + _mm(rv, wr_i_ref[...])
                            + _mm(hh, wh_i_ref[...]) + b_i_ref[...])
        fg = jax.nn.sigmoid(_mm(qq, wq_f_ref[...]) + _mm(rv, wr_f_ref[...])
                            + _mm(hh, wh_f_ref[...]) + b_f_ref[...])
        gg = jnp.tanh(_mm(qq, wq_g_ref[...]) + _mm(rv, wr_g_ref[...])
                      + _mm(hh, wh_g_ref[...]) + b_g_ref[...])
        og = jax.nn.sigmoid(_mm(qq, wq_o_ref[...]) + _mm(rv, wr_o_ref[...])
                            + _mm(hh, wh_o_ref[...]) + b_o_ref[...])
        cc = fg * cc + ig * gg
        hh = og * jnp.tanh(cc)
        q = hh
        qn = _mm(onehot, q)
        e = jnp.sum(M * qn, axis=1, keepdims=True)
        masked = jnp.where(onehot > 0.0, e, -jnp.inf)
        emax = jnp.max(masked, axis=0, keepdims=True)
        emax = jnp.where(emax == -jnp.inf, 0.0, emax)
        emax_n = jnp.sum(onehot * emax, axis=1, keepdims=True)
        ee = jnp.exp(e - emax_n)
        denom = jnp.sum(onehot * ee, axis=0, keepdims=True)
        denom_n = jnp.sum(onehot * denom, axis=1, keepdims=True)
        a = ee / (denom_n + 1e-16)
        rv = _mm(onehot_t, a * M)
        qq = q
    o1 = jnp.maximum(_mm(qq, fc1q_ref[...]) + _mm(rv, fc1r_ref[...]) + fc1b_ref[...], 0.0)
    out_ref[...] = _mm(o1, fc2_ref[...]) + fc2b_ref[...]


def kernel(node_features, edge_index, edge_features, graph_index,
           lin0_W, lin0_b, en1_W, en1_b, en2_W, en2_b, conv_b,
           gru_Wih, gru_Whh, gru_bih, gru_bhh,
           lstm_Wih, lstm_Whh, lstm_bih, lstm_bhh,
           fc1_W, fc1_b, fc2_W, fc2_b):
    f32 = jnp.float32
    src_r = jnp.pad(edge_index[0], (0, EP - E)).reshape(NW, NCHUNK, CH)
    dst_r = jnp.pad(edge_index[1], (0, EP - E), constant_values=N).reshape(NW, NCHUNK, CH)
    ef_pad = jnp.pad(edge_features, ((0, EP - E), (0, 16 - 11)))
    nf_pad = jnp.pad(node_features, ((0, NP - N), (0, 0)))
    gidx_pad = jnp.pad(graph_index, (0, NP - N), constant_values=NG)
    gcol = gidx_pad.reshape(NP, 1)
    grow = gidx_pad.reshape(1, NP)
    ones_msg = jnp.pad(jnp.ones((E, DIM), f32), ((0, EP - E), (0, 0)))
    zeros_np = jnp.zeros((NP, DIM), f32)
    perm = (jnp.arange(DIM * DIM) % DIM) * DIM + jnp.arange(DIM * DIM) // DIM
    k2p = jnp.kron(jnp.eye(DIM, dtype=f32), jnp.ones((DIM, 1), f32))

    en1_Wt = jnp.pad(en1_W.T, ((0, 16 - 11), (0, 0)))
    en2_Wt = en2_W[perm].T
    b1 = en1_b.reshape(1, EH)
    b2 = en2_b[perm].reshape(1, DIM * DIM)
    lin0_Wt = lin0_W.T
    lin0_br = lin0_b.reshape(1, DIM)
    convb = conv_b.reshape(1, DIM)

    wir = gru_Wih[0:DIM].T
    wiz = gru_Wih[DIM:2 * DIM].T
    win = gru_Wih[2 * DIM:3 * DIM].T
    whr = gru_Whh[0:DIM].T
    whz = gru_Whh[DIM:2 * DIM].T
    whn = gru_Whh[2 * DIM:3 * DIM].T
    br = (gru_bih[0:DIM] + gru_bhh[0:DIM]).reshape(1, DIM)
    bz = (gru_bih[DIM:2 * DIM] + gru_bhh[DIM:2 * DIM]).reshape(1, DIM)
    bin_ = gru_bih[2 * DIM:3 * DIM].reshape(1, DIM)
    bhn = gru_bhh[2 * DIM:3 * DIM].reshape(1, DIM)

    def lstm_slices(g):
        lo = g * DIM
        wq = lstm_Wih[lo:lo + DIM, 0:DIM].T
        wr = lstm_Wih[lo:lo + DIM, DIM:2 * DIM].T
        wh = lstm_Whh[lo:lo + DIM].T
        b = (lstm_bih[lo:lo + DIM] + lstm_bhh[lo:lo + DIM]).reshape(1, DIM)
        return wq, wr, wh, b

    lstm_w = [w for g in range(4) for w in lstm_slices(g)]
    fc1q = fc1_W[:, 0:DIM].T
    fc1r = fc1_W[:, DIM:2 * DIM].T
    fc1b = fc1_b.reshape(1, DIM)
    fc2t = fc2_W.T
    fc2b = fc2_b.reshape(1, 1)

    ew = pl.pallas_call(
        _edge_net_body,
        grid=(NEB,),
        in_specs=[
            pl.BlockSpec((EB, 16), lambda i: (i, 0)),
            pl.BlockSpec((16, EH), lambda i: (0, 0)),
            pl.BlockSpec((1, EH), lambda i: (0, 0)),
            pl.BlockSpec((EH, DIM * DIM), lambda i: (0, 0)),
            pl.BlockSpec((1, DIM * DIM), lambda i: (0, 0)),
        ],
        out_specs=pl.BlockSpec((EB, DIM * DIM), lambda i: (i, 0)),
        out_shape=jax.ShapeDtypeStruct((EP, DIM * DIM), f32),
    )(ef_pad, en1_Wt, b1, en2_Wt, b2)

    h = pl.pallas_call(
        _lin0_body,
        grid=(NNB,),
        in_specs=[
            pl.BlockSpec((NB, F_IN), lambda i: (i, 0)),
            pl.BlockSpec((F_IN, DIM), lambda i: (0, 0)),
            pl.BlockSpec((1, DIM), lambda i: (0, 0)),
        ],
        out_specs=pl.BlockSpec((NB, DIM), lambda i: (i, 0)),
        out_shape=jax.ShapeDtypeStruct((NP, DIM), f32),
    )(nf_pad, lin0_Wt, lin0_br)

    degp = _sc_scatter_add(ones_msg, dst_r, zeros_np)

    msg_call = pl.pallas_call(
        _msg_body,
        grid=(NEB,),
        in_specs=[
            pl.BlockSpec((EB, DIM), lambda i: (i, 0)),
            pl.BlockSpec((EB, DIM * DIM), lambda i: (i, 0)),
            pl.BlockSpec((DIM * DIM, DIM), lambda i: (0, 0)),
        ],
        out_specs=pl.BlockSpec((EB, DIM), lambda i: (i, 0)),
        out_shape=jax.ShapeDtypeStruct((EP, DIM), f32),
    )

    _nblk = pl.BlockSpec((NB, DIM), lambda i: (i, 0))
    _wblk = pl.BlockSpec((DIM, DIM), lambda i: (0, 0))
    _bblk = pl.BlockSpec((1, DIM), lambda i: (0, 0))
    gru_call = pl.pallas_call(
        _gru_body,
        grid=(NNB,),
        in_specs=[_nblk] * 5 + [_bblk] + [_wblk] * 6 + [_bblk] * 4,
        out_specs=_nblk,
        out_shape=jax.ShapeDtypeStruct((NP, DIM), f32),
    )

    for _ in range(3):
        xg = _sc_gather(h, src_r)
        msg = msg_call(xg, ew, k2p)
        aggp = _sc_scatter_add(msg, dst_r, zeros_np)
        h = gru_call(aggp[0], aggp[1], degp[0], degp[1], h, convb,
                     wir, wiz, win, whr, whz, whn, br, bz, bin_, bhn)

    pred = pl.pallas_call(
        _set2set_body,
        out_shape=jax.ShapeDtypeStruct((NG, 1), f32),
    )(h, gcol, grow, *lstm_w, fc1q, fc1r, fc1b, fc2t, fc2b)
    return pred.reshape(-1)


# speedup vs baseline: 2.3280x; 1.0159x over previous
import functools

import jax
import jax.numpy as jnp
from jax import lax
from jax.experimental import pallas as pl
from jax.experimental.pallas import tpu as pltpu
from jax.experimental.pallas import tpu_sc as plsc

N = 10000
E = 160000
F_IN = 128
DIM = 16
NG = 64
EH = 128

NC, NS, L = 2, 16, 16
NW = NC * NS
EPT = 5120
EP = NW * EPT
CH = 128
NCHUNK = EPT // CH
GRP = 8
NP = 10240
RPT = NP // NS
EB = 2048
NEB = EP // EB
NB = 1024
NNB = NP // NB

_mesh = plsc.VectorSubcoreMesh(
    core_axis_name="c", subcore_axis_name="s", num_cores=NC, num_subcores=NS)


@functools.partial(
    pl.kernel,
    out_type=jax.ShapeDtypeStruct((EP, DIM), jnp.float32),
    mesh=_mesh,
    compiler_params=pltpu.CompilerParams(use_tc_tiling_on_sc=False),
    scratch_types=[
        pltpu.VMEM((NCHUNK, CH), jnp.int32),
        pltpu.VMEM((EPT, DIM), jnp.float32),
        pltpu.SemaphoreType.DMA,
    ],
)
def _sc_gather(table_hbm, idx_hbm, out_hbm, idx_v, rows_v, sem):
    c = lax.axis_index("c")
    s = lax.axis_index("s")
    wid = s * NC + c
    base = wid * EPT
    pltpu.sync_copy(idx_hbm.at[wid], idx_v)

    def grp_body(g, carry):
        hs = []
        for b in range(GRP):
            j = g * GRP + b
            hs.append(
                pltpu.async_copy(
                    table_hbm.at[idx_v.at[j]],
                    rows_v.at[pl.ds(j * CH, CH)],
                    sem,
                )
            )
        for h in hs:
            h.wait()
        return carry

    lax.fori_loop(0, NCHUNK // GRP, grp_body, 0)
    pltpu.sync_copy(rows_v, out_hbm.at[pl.ds(base, EPT)])


@functools.partial(
    pl.kernel,
    out_type=jax.ShapeDtypeStruct((NC, NP, DIM), jnp.float32),
    mesh=_mesh,
    compiler_params=pltpu.CompilerParams(use_tc_tiling_on_sc=False),
    scratch_types=[
        pltpu.VMEM((NCHUNK, CH), jnp.int32),
        pltpu.VMEM((EPT, DIM), jnp.float32),
        pltpu.VMEM_SHARED((NP, DIM), jnp.float32),
        pltpu.SemaphoreType.DMA,
    ],
)
def _sc_scatter_add(msg_hbm, idx_hbm, zeros_hbm, out_hbm, idx_v, msg_v, acc_sh, sem):
    c = lax.axis_index("c")
    s = lax.axis_index("s")
    wid = s * NC + c
    base = wid * EPT

    @pl.when(s == 0)
    def _():
        pltpu.sync_copy(zeros_hbm, acc_sh)

    plsc.subcore_barrier()
    pltpu.sync_copy(idx_hbm.at[wid], idx_v)
    pltpu.sync_copy(msg_hbm.at[pl.ds(base, EPT)], msg_v)

    def grp_body(g, carry):
        hs = []
        for b in range(GRP):
            j = g * GRP + b
            hs.append(
                pltpu.async_copy(
                    msg_v.at[pl.ds(j * CH, CH)],
                    acc_sh.at[idx_v.at[j]],
                    sem,
                    add=True,
                )
            )
        for h in hs:
            h.wait()
        return carry

    lax.fori_loop(0, NCHUNK // GRP, grp_body, 0)
    plsc.subcore_barrier()
    pltpu.sync_copy(acc_sh.at[pl.ds(s * RPT, RPT)], msg_v.at[pl.ds(0, RPT)])
    pltpu.sync_copy(msg_v.at[pl.ds(0, RPT)], out_hbm.at[c, pl.ds(s * RPT, RPT)])


def _mm(a, b):
    return jnp.matmul(a, b, precision=lax.Precision.HIGHEST)


def _bsplit(a):
    hi = a.astype(jnp.bfloat16).astype(jnp.float32)
    return hi, a - hi


def _mm3(a, b):
    ah, al = _bsplit(a)
    bh, bl = _bsplit(b)
    return jnp.dot(ah, bh) + (jnp.dot(al, bh) + jnp.dot(ah, bl))


def _mm2(a, b):
    ah, al = _bsplit(a)
    return jnp.dot(ah, b) + jnp.dot(al, b)


def _edge_net_body(ef_ref, w1_ref, b1_ref, w2_ref, b2_ref, ew_ref):
    h = jnp.maximum(_mm(ef_ref[...], w1_ref[...]) + b1_ref[...], 0.0)
    ew_ref[...] = _mm(h, w2_ref[...]) + b2_ref[...]


def _lin0_body(nf_ref, w_ref, b_ref, out_ref):
    out_ref[...] = jnp.maximum(_mm(nf_ref[...], w_ref[...]) + b_ref[...], 0.0)


def _msg_body(xg_ref, ew_ref, k1_ref, k2_ref, msg_ref):
    xrep = _mm2(xg_ref[...], k1_ref[...])
    p = xrep * ew_ref[...]
    msg_ref[...] = _mm2(p, k2_ref[...])


def _gru_body(agg0_ref, agg1_ref, deg0_ref, deg1_ref, h_ref, convb_ref,
              wir_ref, wiz_ref, win_ref, whr_ref, whz_ref, whn_ref,
              br_ref, bz_ref, bin_ref, bhn_ref, out_ref):
    deg = jnp.maximum(deg0_ref[...] + deg1_ref[...], 1.0)
    agg = (agg0_ref[...] + agg1_ref[...]) / deg + convb_ref[...]
    m = jnp.maximum(agg, 0.0)
    h = h_ref[...]
    r = jax.nn.sigmoid(_mm(m, wir_ref[...]) + _mm(h, whr_ref[...]) + br_ref[...])
    z = jax.nn.sigmoid(_mm(m, wiz_ref[...]) + _mm(h, whz_ref[...]) + bz_ref[...])
    n = jnp.tanh(_mm(m, win_ref[...]) + bin_ref[...]
                 + r * (_mm(h, whn_ref[...]) + bhn_ref[...]))
    out_ref[...] = (1.0 - z) * n + z * h


def _set2set_body(m_ref, gcol_ref, grow_ref,
                  wq_i_ref, wr_i_ref, wh_i_ref, b_i_ref,
                  wq_f_ref, wr_f_ref, wh_f_ref, b_f_ref,
                  wq_g_ref, wr_g_ref, wh_g_ref, b_g_ref,
                  wq_o_ref, wr_o_ref, wh_o_ref, b_o_ref,
                  fc1q_ref, fc1r_ref, fc1b_ref, fc2_ref, fc2b_ref,
                  out_ref):
    M = m_ref[...]
    gcol = gcol_ref[...]
    grow = grow_ref[...]
    onehot = (gcol == lax.broadcasted_iota(jnp.int32, (NP, NG), 1)
              ).astype(jnp.float32)
    onehot_t = (grow == lax.broadcasted_iota(jnp.int32, (NG, NP), 0)
                ).astype(jnp.float32)

    qq = jnp.zeros((NG, DIM), jnp.float32)
    rv = jnp.zeros((NG, DIM), jnp.float32)
    hh = jnp.zeros((NG, DIM), jnp.float32)
    cc = jnp.zeros((NG, DIM), jnp.float32)
    for _ in range(3):
        ig = jax.nn.sigmoid(_mm(qq, wq_i_ref[...]) + _mm(rv, wr_i_ref[...])
                            + _mm(hh, wh_i_ref[...]) + b_i_ref[...])
        fg = jax.nn.sigmoid(_mm(qq, wq_f_ref[...]) + _mm(rv, wr_f_ref[...])
                            + _mm(hh, wh_f_ref[...]) + b_f_ref[...])
        gg = jnp.tanh(_mm(qq, wq_g_ref[...]) + _mm(rv, wr_g_ref[...])
                      + _mm(hh, wh_g_ref[...]) + b_g_ref[...])
        og = jax.nn.sigmoid(_mm(qq, wq_o_ref[...]) + _mm(rv, wr_o_ref[...])
                            + _mm(hh, wh_o_ref[...]) + b_o_ref[...])
        cc = fg * cc + ig * gg
        hh = og * jnp.tanh(cc)
        q = hh
        qn = _mm(onehot, q)
        e = jnp.sum(M * qn, axis=1, keepdims=True)
        masked = jnp.where(onehot > 0.0, e, -jnp.inf)
        emax = jnp.max(masked, axis=0, keepdims=True)
        emax = jnp.where(emax == -jnp.inf, 0.0, emax)
        emax_n = jnp.sum(onehot * emax, axis=1, keepdims=True)
        ee = jnp.exp(e - emax_n)
        denom = jnp.sum(onehot * ee, axis=0, keepdims=True)
        denom_n = jnp.sum(onehot * denom, axis=1, keepdims=True)
        a = ee / (denom_n + 1e-16)
        rv = _mm(onehot_t, a * M)
        qq = q
    o1 = jnp.maximum(_mm(qq, fc1q_ref[...]) + _mm(rv, fc1r_ref[...]) + fc1b_ref[...], 0.0)
    out_ref[...] = _mm(o1, fc2_ref[...]) + fc2b_ref[...]


def kernel(node_features, edge_index, edge_features, graph_index,
           lin0_W, lin0_b, en1_W, en1_b, en2_W, en2_b, conv_b,
           gru_Wih, gru_Whh, gru_bih, gru_bhh,
           lstm_Wih, lstm_Whh, lstm_bih, lstm_bhh,
           fc1_W, fc1_b, fc2_W, fc2_b):
    f32 = jnp.float32
    src_r = jnp.pad(edge_index[0], (0, EP - E)).reshape(NW, NCHUNK, CH)
    dst_r = jnp.pad(edge_index[1], (0, EP - E), constant_values=N).reshape(NW, NCHUNK, CH)
    ef_pad = jnp.pad(edge_features, ((0, EP - E), (0, 16 - 11)))
    nf_pad = jnp.pad(node_features, ((0, NP - N), (0, 0)))
    gidx_pad = jnp.pad(graph_index, (0, NP - N), constant_values=NG)
    gcol = gidx_pad.reshape(NP, 1)
    grow = gidx_pad.reshape(1, NP)
    ones_msg = jnp.pad(jnp.ones((E, DIM), f32), ((0, EP - E), (0, 0)))
    zeros_np = jnp.zeros((NP, DIM), f32)
    perm = (jnp.arange(DIM * DIM) % DIM) * DIM + jnp.arange(DIM * DIM) // DIM
    k1p = jnp.kron(jnp.ones((1, DIM), f32), jnp.eye(DIM, dtype=f32))
    k2p = jnp.kron(jnp.eye(DIM, dtype=f32), jnp.ones((DIM, 1), f32))

    en1_Wt = jnp.pad(en1_W.T, ((0, 16 - 11), (0, 0)))
    en2_Wt = en2_W[perm].T
    b1 = en1_b.reshape(1, EH)
    b2 = en2_b[perm].reshape(1, DIM * DIM)
    lin0_Wt = lin0_W.T
    lin0_br = lin0_b.reshape(1, DIM)
    convb = conv_b.reshape(1, DIM)

    wir = gru_Wih[0:DIM].T
    wiz = gru_Wih[DIM:2 * DIM].T
    win = gru_Wih[2 * DIM:3 * DIM].T
    whr = gru_Whh[0:DIM].T
    whz = gru_Whh[DIM:2 * DIM].T
    whn = gru_Whh[2 * DIM:3 * DIM].T
    br = (gru_bih[0:DIM] + gru_bhh[0:DIM]).reshape(1, DIM)
    bz = (gru_bih[DIM:2 * DIM] + gru_bhh[DIM:2 * DIM]).reshape(1, DIM)
    bin_ = gru_bih[2 * DIM:3 * DIM].reshape(1, DIM)
    bhn = gru_bhh[2 * DIM:3 * DIM].reshape(1, DIM)

    def lstm_slices(g):
        lo = g * DIM
        wq = lstm_Wih[lo:lo + DIM, 0:DIM].T
        wr = lstm_Wih[lo:lo + DIM, DIM:2 * DIM].T
        wh = lstm_Whh[lo:lo + DIM].T
        b = (lstm_bih[lo:lo + DIM] + lstm_bhh[lo:lo + DIM]).reshape(1, DIM)
        return wq, wr, wh, b

    lstm_w = [w for g in range(4) for w in lstm_slices(g)]
    fc1q = fc1_W[:, 0:DIM].T
    fc1r = fc1_W[:, DIM:2 * DIM].T
    fc1b = fc1_b.reshape(1, DIM)
    fc2t = fc2_W.T
    fc2b = fc2_b.reshape(1, 1)

    ew = pl.pallas_call(
        _edge_net_body,
        grid=(NEB,),
        in_specs=[
            pl.BlockSpec((EB, 16), lambda i: (i, 0)),
            pl.BlockSpec((16, EH), lambda i: (0, 0)),
            pl.BlockSpec((1, EH), lambda i: (0, 0)),
            pl.BlockSpec((EH, DIM * DIM), lambda i: (0, 0)),
            pl.BlockSpec((1, DIM * DIM), lambda i: (0, 0)),
        ],
        out_specs=pl.BlockSpec((EB, DIM * DIM), lambda i: (i, 0)),
        out_shape=jax.ShapeDtypeStruct((EP, DIM * DIM), f32),
    )(ef_pad, en1_Wt, b1, en2_Wt, b2)

    h = pl.pallas_call(
        _lin0_body,
        grid=(NNB,),
        in_specs=[
            pl.BlockSpec((NB, F_IN), lambda i: (i, 0)),
            pl.BlockSpec((F_IN, DIM), lambda i: (0, 0)),
            pl.BlockSpec((1, DIM), lambda i: (0, 0)),
        ],
        out_specs=pl.BlockSpec((NB, DIM), lambda i: (i, 0)),
        out_shape=jax.ShapeDtypeStruct((NP, DIM), f32),
    )(nf_pad, lin0_Wt, lin0_br)

    degp = _sc_scatter_add(ones_msg, dst_r, zeros_np)

    msg_call = pl.pallas_call(
        _msg_body,
        grid=(NEB,),
        in_specs=[
            pl.BlockSpec((EB, DIM), lambda i: (i, 0)),
            pl.BlockSpec((EB, DIM * DIM), lambda i: (i, 0)),
            pl.BlockSpec((DIM, DIM * DIM), lambda i: (0, 0)),
            pl.BlockSpec((DIM * DIM, DIM), lambda i: (0, 0)),
        ],
        out_specs=pl.BlockSpec((EB, DIM), lambda i: (i, 0)),
        out_shape=jax.ShapeDtypeStruct((EP, DIM), f32),
    )

    _nblk = pl.BlockSpec((NB, DIM), lambda i: (i, 0))
    _wblk = pl.BlockSpec((DIM, DIM), lambda i: (0, 0))
    _bblk = pl.BlockSpec((1, DIM), lambda i: (0, 0))
    gru_call = pl.pallas_call(
        _gru_body,
        grid=(NNB,),
        in_specs=[_nblk] * 5 + [_bblk] + [_wblk] * 6 + [_bblk] * 4,
        out_specs=_nblk,
        out_shape=jax.ShapeDtypeStruct((NP, DIM), f32),
    )

    for _ in range(3):
        xg = _sc_gather(h, src_r)
        msg = msg_call(xg, ew, k1p, k2p)
        aggp = _sc_scatter_add(msg, dst_r, zeros_np)
        h = gru_call(aggp[0], aggp[1], degp[0], degp[1], h, convb,
                     wir, wiz, win, whr, whz, whn, br, bz, bin_, bhn)

    pred = pl.pallas_call(
        _set2set_body,
        out_shape=jax.ShapeDtypeStruct((NG, 1), f32),
    )(h, gcol, grow, *lstm_w, fc1q, fc1r, fc1b, fc2t, fc2b)
    return pred.reshape(-1)


# speedup vs baseline: 2.3775x; 1.0213x over previous
import functools

import jax
import jax.numpy as jnp
from jax import lax
from jax.experimental import pallas as pl
from jax.experimental.pallas import tpu as pltpu
from jax.experimental.pallas import tpu_sc as plsc

N = 10000
E = 160000
F_IN = 128
DIM = 16
NG = 64
EH = 128

NC, NS, L = 2, 16, 16
NW = NC * NS
EPT = 5120
EP = NW * EPT
CH = 128
NCHUNK = EPT // CH
GRP = 8
NP = 10240
RPT = NP // NS
EB = 2048
NEB = EP // EB
NB = 1024
NNB = NP // NB

_mesh = plsc.VectorSubcoreMesh(
    core_axis_name="c", subcore_axis_name="s", num_cores=NC, num_subcores=NS)


@functools.partial(
    pl.kernel,
    out_type=jax.ShapeDtypeStruct((EP, DIM), jnp.float32),
    mesh=_mesh,
    compiler_params=pltpu.CompilerParams(use_tc_tiling_on_sc=False),
    scratch_types=[
        pltpu.VMEM((NCHUNK, CH), jnp.int32),
        pltpu.VMEM((EPT, DIM), jnp.float32),
        pltpu.SemaphoreType.DMA,
    ],
)
def _sc_gather(table_hbm, idx_hbm, out_hbm, idx_v, rows_v, sem):
    c = lax.axis_index("c")
    s = lax.axis_index("s")
    wid = s * NC + c
    base = wid * EPT
    pltpu.sync_copy(idx_hbm.at[wid], idx_v)

    def grp_body(g, carry):
        hs = []
        for b in range(GRP):
            j = g * GRP + b
            hs.append(
                pltpu.async_copy(
                    table_hbm.at[idx_v.at[j]],
                    rows_v.at[pl.ds(j * CH, CH)],
                    sem,
                )
            )
        for h in hs:
            h.wait()
        return carry

    lax.fori_loop(0, NCHUNK // GRP, grp_body, 0)
    pltpu.sync_copy(rows_v, out_hbm.at[pl.ds(base, EPT)])


@functools.partial(
    pl.kernel,
    out_type=jax.ShapeDtypeStruct((NC, NP, DIM), jnp.float32),
    mesh=_mesh,
    compiler_params=pltpu.CompilerParams(use_tc_tiling_on_sc=False),
    scratch_types=[
        pltpu.VMEM((NCHUNK, CH), jnp.int32),
        pltpu.VMEM((EPT, DIM), jnp.float32),
        pltpu.VMEM_SHARED((NP, DIM), jnp.float32),
        pltpu.SemaphoreType.DMA,
    ],
)
def _sc_scatter_add(msg_hbm, idx_hbm, zeros_hbm, out_hbm, idx_v, msg_v, acc_sh, sem):
    c = lax.axis_index("c")
    s = lax.axis_index("s")
    wid = s * NC + c
    base = wid * EPT

    @pl.when(s == 0)
    def _():
        pltpu.sync_copy(zeros_hbm, acc_sh)

    plsc.subcore_barrier()
    pltpu.sync_copy(idx_hbm.at[wid], idx_v)
    pltpu.sync_copy(msg_hbm.at[pl.ds(base, EPT)], msg_v)

    def grp_body(g, carry):
        hs = []
        for b in range(GRP):
            j = g * GRP + b
            hs.append(
                pltpu.async_copy(
                    msg_v.at[pl.ds(j * CH, CH)],
                    acc_sh.at[idx_v.at[j]],
                    sem,
                    add=True,
                )
            )
        for h in hs:
            h.wait()
        return carry

    lax.fori_loop(0, NCHUNK // GRP, grp_body, 0)
    plsc.subcore_barrier()
    pltpu.sync_copy(acc_sh.at[pl.ds(s * RPT, RPT)], msg_v.at[pl.ds(0, RPT)])
    pltpu.sync_copy(msg_v.at[pl.ds(0, RPT)], out_hbm.at[c, pl.ds(s * RPT, RPT)])


def _mm(a, b):
    return jnp.matmul(a, b, precision=lax.Precision.HIGHEST)


def _bsplit(a):
    hi = a.astype(jnp.bfloat16).astype(jnp.float32)
    return hi, a - hi


def _mm3(a, b):
    ah, al = _bsplit(a)
    bh, bl = _bsplit(b)
    return jnp.dot(ah, bh) + (jnp.dot(al, bh) + jnp.dot(ah, bl))


def _mm2(a, b):
    ah, al = _bsplit(a)
    return jnp.dot(ah, b) + jnp.dot(al, b)


def _edge_net_body(ef_ref, w1_ref, b1_ref, w2_ref, b2_ref, ew_ref):
    h = jnp.maximum(_mm(ef_ref[...], w1_ref[...]) + b1_ref[...], 0.0)
    ew_ref[...] = (_mm(h, w2_ref[...]) + b2_ref[...]).astype(jnp.bfloat16)


def _lin0_body(nf_ref, w_ref, b_ref, out_ref):
    out_ref[...] = jnp.maximum(_mm(nf_ref[...], w_ref[...]) + b_ref[...], 0.0)


def _msg_body(xg_ref, ew_ref, k1_ref, k2_ref, msg_ref):
    xrep = _mm2(xg_ref[...], k1_ref[...])
    p = xrep * ew_ref[...].astype(jnp.float32)
    msg_ref[...] = _mm2(p, k2_ref[...])


def _gru_body(agg0_ref, agg1_ref, deg0_ref, deg1_ref, h_ref, convb_ref,
              wir_ref, wiz_ref, win_ref, whr_ref, whz_ref, whn_ref,
              br_ref, bz_ref, bin_ref, bhn_ref, out_ref):
    deg = jnp.maximum(deg0_ref[...] + deg1_ref[...], 1.0)
    agg = (agg0_ref[...] + agg1_ref[...]) / deg + convb_ref[...]
    m = jnp.maximum(agg, 0.0)
    h = h_ref[...]
    r = jax.nn.sigmoid(_mm(m, wir_ref[...]) + _mm(h, whr_ref[...]) + br_ref[...])
    z = jax.nn.sigmoid(_mm(m, wiz_ref[...]) + _mm(h, whz_ref[...]) + bz_ref[...])
    n = jnp.tanh(_mm(m, win_ref[...]) + bin_ref[...]
                 + r * (_mm(h, whn_ref[...]) + bhn_ref[...]))
    out_ref[...] = (1.0 - z) * n + z * h


def _set2set_body(m_ref, gcol_ref, grow_ref,
                  wq_i_ref, wr_i_ref, wh_i_ref, b_i_ref,
                  wq_f_ref, wr_f_ref, wh_f_ref, b_f_ref,
                  wq_g_ref, wr_g_ref, wh_g_ref, b_g_ref,
                  wq_o_ref, wr_o_ref, wh_o_ref, b_o_ref,
                  fc1q_ref, fc1r_ref, fc1b_ref, fc2_ref, fc2b_ref,
                  out_ref):
    M = m_ref[...]
    gcol = gcol_ref[...]
    grow = grow_ref[...]
    onehot = (gcol == lax.broadcasted_iota(jnp.int32, (NP, NG), 1)
              ).astype(jnp.float32)
    onehot_t = (grow == lax.broadcasted_iota(jnp.int32, (NG, NP), 0)
                ).astype(jnp.float32)

    qq = jnp.zeros((NG, DIM), jnp.float32)
    rv = jnp.zeros((NG, DIM), jnp.float32)
    hh = jnp.zeros((NG, DIM), jnp.float32)
    cc = jnp.zeros((NG, DIM), jnp.float32)
    for _ in range(3):
        ig = jax.nn.sigmoid(_mm(qq, wq_i_ref[...]) + _mm(rv, wr_i_ref[...])
                            + _mm(hh, wh_i_ref[...]) + b_i_ref[...])
        fg = jax.nn.sigmoid(_mm(qq, wq_f_ref[...]) + _mm(rv, wr_f_ref[...])
                            + _mm(hh, wh_f_ref[...]) + b_f_ref[...])
        gg = jnp.tanh(_mm(qq, wq_g_ref[...]) + _mm(rv, wr_g_ref[...])
                      + _mm(hh, wh_g_ref[...]) + b_g_ref[...])
        og = jax.nn.sigmoid(_mm(qq, wq_o_ref[...]) + _mm(rv, wr_o_ref[...])
                            + _mm(hh, wh_o_ref[...]) + b_o_ref[...])
        cc = fg * cc + ig * gg
        hh = og * jnp.tanh(cc)
        q = hh
        qn = _mm(onehot, q)
        e = jnp.sum(M * qn, axis=1, keepdims=True)
        masked = jnp.where(onehot > 0.0, e, -jnp.inf)
        emax = jnp.max(masked, axis=0, keepdims=True)
        emax = jnp.where(emax == -jnp.inf, 0.0, emax)
        emax_n = jnp.sum(onehot * emax, axis=1, keepdims=True)
        ee = jnp.exp(e - emax_n)
        denom = jnp.sum(onehot * ee, axis=0, keepdims=True)
        denom_n = jnp.sum(onehot * denom, axis=1, keepdims=True)
        a = ee / (denom_n + 1e-16)
        rv = _mm(onehot_t, a * M)
        qq = q
    o1 = jnp.maximum(_mm(qq, fc1q_ref[...]) + _mm(rv, fc1r_ref[...]) + fc1b_ref[...], 0.0)
    out_ref[...] = _mm(o1, fc2_ref[...]) + fc2b_ref[...]


def kernel(node_features, edge_index, edge_features, graph_index,
           lin0_W, lin0_b, en1_W, en1_b, en2_W, en2_b, conv_b,
           gru_Wih, gru_Whh, gru_bih, gru_bhh,
           lstm_Wih, lstm_Whh, lstm_bih, lstm_bhh,
           fc1_W, fc1_b, fc2_W, fc2_b):
    f32 = jnp.float32
    src_r = jnp.pad(edge_index[0], (0, EP - E)).reshape(NW, NCHUNK, CH)
    dst_r = jnp.pad(edge_index[1], (0, EP - E), constant_values=N).reshape(NW, NCHUNK, CH)
    ef_pad = jnp.pad(edge_features, ((0, EP - E), (0, 16 - 11)))
    nf_pad = jnp.pad(node_features, ((0, NP - N), (0, 0)))
    gidx_pad = jnp.pad(graph_index, (0, NP - N), constant_values=NG)
    gcol = gidx_pad.reshape(NP, 1)
    grow = gidx_pad.reshape(1, NP)
    ones_msg = jnp.pad(jnp.ones((E, DIM), f32), ((0, EP - E), (0, 0)))
    zeros_np = jnp.zeros((NP, DIM), f32)
    perm = (jnp.arange(DIM * DIM) % DIM) * DIM + jnp.arange(DIM * DIM) // DIM
    k1p = jnp.kron(jnp.ones((1, DIM), f32), jnp.eye(DIM, dtype=f32))
    k2p = jnp.kron(jnp.eye(DIM, dtype=f32), jnp.ones((DIM, 1), f32))

    en1_Wt = jnp.pad(en1_W.T, ((0, 16 - 11), (0, 0)))
    en2_Wt = en2_W[perm].T
    b1 = en1_b.reshape(1, EH)
    b2 = en2_b[perm].reshape(1, DIM * DIM)
    lin0_Wt = lin0_W.T
    lin0_br = lin0_b.reshape(1, DIM)
    convb = conv_b.reshape(1, DIM)

    wir = gru_Wih[0:DIM].T
    wiz = gru_Wih[DIM:2 * DIM].T
    win = gru_Wih[2 * DIM:3 * DIM].T
    whr = gru_Whh[0:DIM].T
    whz = gru_Whh[DIM:2 * DIM].T
    whn = gru_Whh[2 * DIM:3 * DIM].T
    br = (gru_bih[0:DIM] + gru_bhh[0:DIM]).reshape(1, DIM)
    bz = (gru_bih[DIM:2 * DIM] + gru_bhh[DIM:2 * DIM]).reshape(1, DIM)
    bin_ = gru_bih[2 * DIM:3 * DIM].reshape(1, DIM)
    bhn = gru_bhh[2 * DIM:3 * DIM].reshape(1, DIM)

    def lstm_slices(g):
        lo = g * DIM
        wq = lstm_Wih[lo:lo + DIM, 0:DIM].T
        wr = lstm_Wih[lo:lo + DIM, DIM:2 * DIM].T
        wh = lstm_Whh[lo:lo + DIM].T
        b = (lstm_bih[lo:lo + DIM] + lstm_bhh[lo:lo + DIM]).reshape(1, DIM)
        return wq, wr, wh, b

    lstm_w = [w for g in range(4) for w in lstm_slices(g)]
    fc1q = fc1_W[:, 0:DIM].T
    fc1r = fc1_W[:, DIM:2 * DIM].T
    fc1b = fc1_b.reshape(1, DIM)
    fc2t = fc2_W.T
    fc2b = fc2_b.reshape(1, 1)

    ew = pl.pallas_call(
        _edge_net_body,
        grid=(NEB,),
        in_specs=[
            pl.BlockSpec((EB, 16), lambda i: (i, 0)),
            pl.BlockSpec((16, EH), lambda i: (0, 0)),
            pl.BlockSpec((1, EH), lambda i: (0, 0)),
            pl.BlockSpec((EH, DIM * DIM), lambda i: (0, 0)),
            pl.BlockSpec((1, DIM * DIM), lambda i: (0, 0)),
        ],
        out_specs=pl.BlockSpec((EB, DIM * DIM), lambda i: (i, 0)),
        out_shape=jax.ShapeDtypeStruct((EP, DIM * DIM), jnp.bfloat16),
    )(ef_pad, en1_Wt, b1, en2_Wt, b2)

    h = pl.pallas_call(
        _lin0_body,
        grid=(NNB,),
        in_specs=[
            pl.BlockSpec((NB, F_IN), lambda i: (i, 0)),
            pl.BlockSpec((F_IN, DIM), lambda i: (0, 0)),
            pl.BlockSpec((1, DIM), lambda i: (0, 0)),
        ],
        out_specs=pl.BlockSpec((NB, DIM), lambda i: (i, 0)),
        out_shape=jax.ShapeDtypeStruct((NP, DIM), f32),
    )(nf_pad, lin0_Wt, lin0_br)

    degp = _sc_scatter_add(ones_msg, dst_r, zeros_np)

    msg_call = pl.pallas_call(
        _msg_body,
        grid=(NEB,),
        in_specs=[
            pl.BlockSpec((EB, DIM), lambda i: (i, 0)),
            pl.BlockSpec((EB, DIM * DIM), lambda i: (i, 0)),
            pl.BlockSpec((DIM, DIM * DIM), lambda i: (0, 0)),
            pl.BlockSpec((DIM * DIM, DIM), lambda i: (0, 0)),
        ],
        out_specs=pl.BlockSpec((EB, DIM), lambda i: (i, 0)),
        out_shape=jax.ShapeDtypeStruct((EP, DIM), f32),
    )

    _nblk = pl.BlockSpec((NB, DIM), lambda i: (i, 0))
    _wblk = pl.BlockSpec((DIM, DIM), lambda i: (0, 0))
    _bblk = pl.BlockSpec((1, DIM), lambda i: (0, 0))
    gru_call = pl.pallas_call(
        _gru_body,
        grid=(NNB,),
        in_specs=[_nblk] * 5 + [_bblk] + [_wblk] * 6 + [_bblk] * 4,
        out_specs=_nblk,
        out_shape=jax.ShapeDtypeStruct((NP, DIM), f32),
    )

    for _ in range(3):
        xg = _sc_gather(h, src_r)
        msg = msg_call(xg, ew, k1p, k2p)
        aggp = _sc_scatter_add(msg, dst_r, zeros_np)
        h = gru_call(aggp[0], aggp[1], degp[0], degp[1], h, convb,
                     wir, wiz, win, whr, whz, whn, br, bz, bin_, bhn)

    pred = pl.pallas_call(
        _set2set_body,
        out_shape=jax.ShapeDtypeStruct((NG, 1), f32),
    )(h, gcol, grow, *lstm_w, fc1q, fc1r, fc1b, fc2t, fc2b)
    return pred.reshape(-1)
